# baseline (device time: 48213 ns/iter reference)
import jax
import jax.numpy as jnp
from jax import lax
from jax.experimental import pallas as pl
from jax.experimental.pallas import tpu as pltpu

N_DEV = 4
N_EXPERTS = 16
CAPACITY = 25
E_LOCAL = N_EXPERTS // N_DEV


def kernel(x, router_W, route_idx, expert_W):
    n, d = x.shape
    h = expert_W.shape[-1]

    def body(x_ref, idx_ref, w_ref, out_ref, comm_ref, send_sems, recv_sems):
        my_pos = lax.axis_index("i")
        left = lax.rem(my_pos + N_DEV - 1, N_DEV)
        right = lax.rem(my_pos + 1, N_DEV)

        barrier_sem = pltpu.get_barrier_semaphore()
        for nbr in (left, right):
            pl.semaphore_signal(
                barrier_sem, inc=1,
                device_id=(nbr,), device_id_type=pl.DeviceIdType.MESH,
            )
        pl.semaphore_wait(barrier_sem, 2)

        e = idx_ref[:, :]
        expert_ids = lax.broadcasted_iota(jnp.int32, (n, N_EXPERTS), 1)
        one_hot = (e == expert_ids).astype(jnp.float32)
        row = lax.broadcasted_iota(jnp.int32, (n, n), 0)
        col = lax.broadcasted_iota(jnp.int32, (n, n), 1)
        strict_lower = (row > col).astype(jnp.float32)
        cum = jnp.dot(strict_lower, one_hot,
                      preferred_element_type=jnp.float32)
        rank = jnp.sum(one_hot * cum, axis=1, keepdims=True)
        keep = rank < float(CAPACITY)

        xv = x_ref[:, :]
        partial = jnp.zeros((n, h), jnp.float32)
        for le in range(E_LOCAL):
            ge = my_pos * E_LOCAL + le
            mask = jnp.logical_and(e == ge, keep).astype(jnp.float32)
            partial = partial + jnp.dot(
                xv * mask, w_ref[le], preferred_element_type=jnp.float32
            )

        out_ref[:, :] = partial
        comm_ref[0, :, :] = partial

        for hop in range(N_DEV - 1):
            rdma = pltpu.make_async_remote_copy(
                src_ref=comm_ref.at[hop],
                dst_ref=comm_ref.at[hop + 1],
                send_sem=send_sems.at[hop],
                recv_sem=recv_sems.at[hop],
                device_id=(right,),
                device_id_type=pl.DeviceIdType.MESH,
            )
            rdma.start()
            rdma.wait()
            out_ref[:, :] = out_ref[:, :] + comm_ref[hop + 1, :, :]

    return pl.pallas_call(
        body,
        out_shape=jax.ShapeDtypeStruct((n, h), jnp.float32),
        in_specs=[
            pl.BlockSpec(memory_space=pltpu.VMEM),
            pl.BlockSpec(memory_space=pltpu.VMEM),
            pl.BlockSpec(memory_space=pltpu.VMEM),
        ],
        out_specs=pl.BlockSpec(memory_space=pltpu.VMEM),
        scratch_shapes=[
            pltpu.VMEM((N_DEV, n, h), jnp.float32),
            pltpu.SemaphoreType.DMA((N_DEV - 1,)),
            pltpu.SemaphoreType.DMA((N_DEV - 1,)),
        ],
        compiler_params=pltpu.CompilerParams(collective_id=0),
    )(x, route_idx, expert_W)


# device time: 17241 ns/iter; 2.7964x vs baseline; 2.7964x over previous
import jax
import jax.numpy as jnp
from jax import lax
from jax.experimental import pallas as pl
from jax.experimental.pallas import tpu as pltpu

N_DEV = 4
N_EXPERTS = 16
CAPACITY = 25
E_LOCAL = N_EXPERTS // N_DEV
SLOT_PER_E = 32
N_SLOTS = E_LOCAL * SLOT_PER_E


def kernel(x, router_W, route_idx, expert_W):
    n, d = x.shape
    h = expert_W.shape[-1]

    def body(x_ref, idx_ref, w_ref, out_ref, comm_ref, send_sems, recv_sems):
        my_pos = lax.axis_index("i")

        barrier_sem = pltpu.get_barrier_semaphore()
        for k in range(1, N_DEV):
            pl.semaphore_signal(
                barrier_sem, inc=1,
                device_id=(lax.rem(my_pos + k, N_DEV),),
                device_id_type=pl.DeviceIdType.MESH,
            )
        pl.semaphore_wait(barrier_sem, N_DEV - 1)

        e = idx_ref[:, :]
        expert_ids = lax.broadcasted_iota(jnp.int32, (n, N_EXPERTS), 1)
        one_hot = (e == expert_ids).astype(jnp.float32)
        row = lax.broadcasted_iota(jnp.int32, (n, n), 0)
        col = lax.broadcasted_iota(jnp.int32, (n, n), 1)
        strict_lower = (row > col).astype(jnp.float32)
        cum = jnp.dot(strict_lower, one_hot,
                      preferred_element_type=jnp.float32)
        rank = jnp.sum(one_hot * cum, axis=1, keepdims=True)
        rank_i = rank.astype(jnp.int32)
        keep = rank_i < CAPACITY
        owner = lax.div(e, E_LOCAL)
        slot = lax.rem(e, E_LOCAL) * SLOT_PER_E + rank_i

        slot_ids = lax.broadcasted_iota(jnp.int32, (n, N_SLOTS), 1)

        def q_mat(c):
            valid = jnp.logical_and(keep, owner == c)
            return jnp.logical_and(slot_ids == slot, valid).astype(jnp.float32)

        q_me = q_mat(my_pos)
        xg = lax.dot_general(
            q_me, x_ref[:, :], (((0,), (0,)), ((), ())),
            preferred_element_type=jnp.float32,
        )
        for le in range(E_LOCAL):
            comm_ref[my_pos, pl.ds(le * SLOT_PER_E, SLOT_PER_E), :] = jnp.dot(
                xg[le * SLOT_PER_E:(le + 1) * SLOT_PER_E],
                w_ref[le],
                preferred_element_type=jnp.float32,
            )

        sends = []
        for k in range(1, N_DEV):
            dst = lax.rem(my_pos + k, N_DEV)
            rdma = pltpu.make_async_remote_copy(
                src_ref=comm_ref.at[my_pos],
                dst_ref=comm_ref.at[my_pos],
                send_sem=send_sems.at[k - 1],
                recv_sem=recv_sems.at[my_pos],
                device_id=(dst,),
                device_id_type=pl.DeviceIdType.MESH,
            )
            rdma.start()
            sends.append(rdma)

        out_ref[:, :] = jnp.dot(q_me, comm_ref[my_pos],
                                preferred_element_type=jnp.float32)

        for k in range(1, N_DEV):
            src = lax.rem(my_pos + k, N_DEV)
            recv = pltpu.make_async_remote_copy(
                src_ref=comm_ref.at[src],
                dst_ref=comm_ref.at[src],
                send_sem=send_sems.at[k - 1],
                recv_sem=recv_sems.at[src],
                device_id=(src,),
                device_id_type=pl.DeviceIdType.MESH,
            )
            recv.wait_recv()
            out_ref[:, :] = out_ref[:, :] + jnp.dot(
                q_mat(src), comm_ref[src], preferred_element_type=jnp.float32
            )

        for rdma in sends:
            rdma.wait_send()

    return pl.pallas_call(
        body,
        out_shape=jax.ShapeDtypeStruct((n, h), jnp.float32),
        in_specs=[
            pl.BlockSpec(memory_space=pltpu.VMEM),
            pl.BlockSpec(memory_space=pltpu.VMEM),
            pl.BlockSpec(memory_space=pltpu.VMEM),
        ],
        out_specs=pl.BlockSpec(memory_space=pltpu.VMEM),
        scratch_shapes=[
            pltpu.VMEM((N_DEV, N_SLOTS, h), jnp.float32),
            pltpu.SemaphoreType.DMA((N_DEV - 1,)),
            pltpu.SemaphoreType.DMA((N_DEV,)),
        ],
        compiler_params=pltpu.CompilerParams(collective_id=0),
    )(x, route_idx, expert_W)


# device time: 13133 ns/iter; 3.6711x vs baseline; 1.3128x over previous
import jax
import jax.numpy as jnp
from jax import lax
from jax.experimental import pallas as pl
from jax.experimental.pallas import tpu as pltpu

N_DEV = 4
N_EXPERTS = 16
CAPACITY = 25
E_LOCAL = N_EXPERTS // N_DEV
SLOT_PER_E = 32
N_SLOTS = E_LOCAL * SLOT_PER_E
G_SLOTS = N_DEV * N_SLOTS


def kernel(x, router_W, route_idx, expert_W):
    n, d = x.shape
    h = expert_W.shape[-1]

    def body(x_ref, idx_ref, w_ref, out_ref, comm_ref, send_sems, recv_sems):
        my_pos = lax.axis_index("i")

        barrier_sem = pltpu.get_barrier_semaphore()
        for k in range(1, N_DEV):
            pl.semaphore_signal(
                barrier_sem, inc=1,
                device_id=(lax.rem(my_pos + k, N_DEV),),
                device_id_type=pl.DeviceIdType.MESH,
            )

        e = idx_ref[:, :]
        expert_ids = lax.broadcasted_iota(jnp.int32, (n, N_EXPERTS), 1)
        one_hot = (e == expert_ids).astype(jnp.bfloat16)
        row = lax.broadcasted_iota(jnp.int32, (n, n), 0)
        col = lax.broadcasted_iota(jnp.int32, (n, n), 1)
        strict_lower = (row > col).astype(jnp.bfloat16)
        cum = jnp.dot(strict_lower, one_hot,
                      preferred_element_type=jnp.float32)
        rank_i = jnp.sum(one_hot.astype(jnp.float32) * cum,
                         axis=1, keepdims=True).astype(jnp.int32)
        keep = rank_i < CAPACITY
        owner = lax.div(e, E_LOCAL)
        slot = lax.rem(e, E_LOCAL) * SLOT_PER_E + rank_i

        slot_ids = lax.broadcasted_iota(jnp.int32, (n, N_SLOTS), 1)
        valid_me = jnp.logical_and(keep, owner == my_pos)
        q_me = jnp.logical_and(slot_ids == slot, valid_me).astype(jnp.bfloat16)
        xg = lax.dot_general(
            q_me, x_ref[:, :].astype(jnp.bfloat16),
            (((0,), (0,)), ((), ())),
            preferred_element_type=jnp.float32,
        ).astype(jnp.bfloat16)

        col_blk = lax.broadcasted_iota(jnp.int32, (N_SLOTS, E_LOCAL * d), 1) // d
        row_blk = lax.broadcasted_iota(
            jnp.int32, (N_SLOTS, E_LOCAL * d), 0) // SLOT_PER_E
        blk_mask = (col_blk == row_blk).astype(jnp.bfloat16)
        xg_wide = jnp.concatenate([xg] * E_LOCAL, axis=1) * blk_mask
        w_flat = w_ref[:, :, :].astype(jnp.bfloat16).reshape(E_LOCAL * d, h)
        yg = jnp.dot(xg_wide, w_flat,
                     preferred_element_type=jnp.float32)
        comm_ref[my_pos, :, :] = yg.astype(jnp.bfloat16)

        pl.semaphore_wait(barrier_sem, N_DEV - 1)
        sends = []
        for k in range(1, N_DEV):
            dst = lax.rem(my_pos + k, N_DEV)
            rdma = pltpu.make_async_remote_copy(
                src_ref=comm_ref.at[my_pos],
                dst_ref=comm_ref.at[my_pos],
                send_sem=send_sems.at[k - 1],
                recv_sem=recv_sems.at[my_pos],
                device_id=(dst,),
                device_id_type=pl.DeviceIdType.MESH,
            )
            rdma.start()
            sends.append(rdma)

        out_ref[:, :] = jnp.dot(q_me, comm_ref[my_pos],
                                preferred_element_type=jnp.float32)

        def q_from(c):
            valid = jnp.logical_and(keep, owner == c)
            return jnp.logical_and(slot_ids == slot, valid).astype(jnp.bfloat16)

        order = [1, 3, 2]
        q_srcs = {k: q_from(lax.rem(my_pos + k, N_DEV)) for k in order}

        for k in order:
            src = lax.rem(my_pos + k, N_DEV)
            recv = pltpu.make_async_remote_copy(
                src_ref=comm_ref.at[src],
                dst_ref=comm_ref.at[src],
                send_sem=send_sems.at[k - 1],
                recv_sem=recv_sems.at[src],
                device_id=(src,),
                device_id_type=pl.DeviceIdType.MESH,
            )
            recv.wait_recv()
            out_ref[:, :] = out_ref[:, :] + jnp.dot(
                q_srcs[k], comm_ref[src], preferred_element_type=jnp.float32
            )

        for rdma in sends:
            rdma.wait_send()

    return pl.pallas_call(
        body,
        out_shape=jax.ShapeDtypeStruct((n, h), jnp.float32),
        in_specs=[
            pl.BlockSpec(memory_space=pltpu.VMEM),
            pl.BlockSpec(memory_space=pltpu.VMEM),
            pl.BlockSpec(memory_space=pltpu.VMEM),
        ],
        out_specs=pl.BlockSpec(memory_space=pltpu.VMEM),
        scratch_shapes=[
            pltpu.VMEM((N_DEV, N_SLOTS, h), jnp.bfloat16),
            pltpu.SemaphoreType.DMA((N_DEV - 1,)),
            pltpu.SemaphoreType.DMA((N_DEV,)),
        ],
        compiler_params=pltpu.CompilerParams(collective_id=0),
    )(x, route_idx, expert_W)
